# baseline (speedup 1.0000x reference)
import functools

import jax
import jax.numpy as jnp
from jax import lax
from jax.experimental import pallas as pl
from jax.experimental.pallas import tpu as pltpu
from jax.experimental.pallas import tpu_sc as plsc

NC = 2
NS = 16
NW = NC * NS
C = 100
NBUF = 5


def _deg_kernel(n, ch):
    rpt = n // NS

    mesh = plsc.VectorSubcoreMesh(core_axis_name="c", subcore_axis_name="s")

    @functools.partial(
        pl.kernel,
        out_type=jax.ShapeDtypeStruct((NC, n, 8), jnp.float32),
        mesh=mesh,
        compiler_params=pltpu.CompilerParams(use_tc_tiling_on_sc=False),
        scratch_types=[
            pltpu.VMEM((ch, C), jnp.int32),
            pltpu.VMEM((C, 16), jnp.float32),
            pltpu.VMEM_SHARED((n, 16), jnp.float32),
        ],
    )
    def k(dst_hbm, zeros_hbm, ones_hbm, out_hbm, dst_v, ones_v, acc):
        c = lax.axis_index("c")
        s = lax.axis_index("s")
        wid = c * NS + s
        pltpu.sync_copy(dst_hbm.at[wid], dst_v)
        pltpu.sync_copy(ones_hbm, ones_v)
        pltpu.sync_copy(zeros_hbm.at[pl.ds(s * rpt, rpt)], acc.at[pl.ds(s * rpt, rpt)])
        plsc.subcore_barrier()

        def body(g, carry):
            pltpu.sync_copy(ones_v, acc.at[dst_v.at[g]], add=True)
            return carry

        lax.fori_loop(0, ch, body, 0)
        plsc.subcore_barrier()
        pltpu.sync_copy(acc.at[pl.ds(s * rpt, rpt), pl.ds(0, 8)],
                        out_hbm.at[c, pl.ds(s * rpt, rpt)])

    return k


def _agg_kernel(n, ch):
    rpt = n // NS

    mesh = plsc.VectorSubcoreMesh(core_axis_name="c", subcore_axis_name="s")

    @functools.partial(
        pl.kernel,
        out_type=jax.ShapeDtypeStruct((NC, n, 16), jnp.float32),
        mesh=mesh,
        compiler_params=pltpu.CompilerParams(use_tc_tiling_on_sc=False),
        scratch_types=[
            pltpu.VMEM((ch, C), jnp.int32),
            pltpu.VMEM((ch, C), jnp.int32),
            pltpu.VMEM((2, NBUF, C, 16), jnp.float32),
            pltpu.VMEM_SHARED((n, 16), jnp.float32),
            pltpu.SemaphoreType.DMA((2,)),
        ],
    )
    def k(hs_hbm, src_hbm, dst_hbm, zeros_hbm, out_hbm,
          src_v, dst_v, rows_v, acc, gsem):
        c = lax.axis_index("c")
        s = lax.axis_index("s")
        wid = c * NS + s
        pltpu.sync_copy(src_hbm.at[wid], src_v)
        pltpu.sync_copy(dst_hbm.at[wid], dst_v)
        pltpu.sync_copy(zeros_hbm.at[pl.ds(s * rpt, rpt)], acc.at[pl.ds(s * rpt, rpt)])
        plsc.subcore_barrier()

        ngroups = ch // NBUF
        npairs = ngroups // 2

        def fire(grp_idx, p):
            for b in range(NBUF):
                pltpu.async_copy(hs_hbm.at[src_v.at[grp_idx * NBUF + b]],
                                 rows_v.at[p, b], gsem.at[p])

        def drain_scatter(grp_idx, p):
            for b in range(NBUF):
                pltpu.make_async_copy(hs_hbm.at[src_v.at[grp_idx * NBUF + b]],
                                      rows_v.at[p, b], gsem.at[p]).wait()
            for b in range(NBUF):
                pltpu.sync_copy(rows_v.at[p, b],
                                acc.at[dst_v.at[grp_idx * NBUF + b]], add=True)

        fire(0, 0)
        fire(1, 1)

        def pair(k_, carry):
            for p in range(2):
                gi = 2 * k_ + p
                drain_scatter(gi, p)
                fire(gi + 2, p)
            return carry

        lax.fori_loop(0, npairs - 1, pair, 0)
        for p in range(2):
            drain_scatter(2 * (npairs - 1) + p, p)
        plsc.subcore_barrier()
        pltpu.sync_copy(acc.at[pl.ds(s * rpt, rpt)],
                        out_hbm.at[c, pl.ds(s * rpt, rpt)])

    return k


def _tc_first(x_ref, w1_ref, pdeg_ref, hs1_ref, dis_ref):
    deg = pdeg_ref[0, :, 0:1] + pdeg_ref[1, :, 0:1] + 1.0
    dis = lax.rsqrt(deg)
    h1 = jnp.dot(x_ref[...], w1_ref[...], preferred_element_type=jnp.float32)
    hs1_ref[...] = h1 * dis
    dis_ref[...] = dis


def _tc_mid(p_ref, hs1_ref, dis_ref, b1_ref, w2_ref, hs2_ref):
    dis = dis_ref[...]
    agg = (p_ref[0] + p_ref[1] + hs1_ref[...]) * dis + b1_ref[...]
    o1 = jnp.maximum(agg, 0.0)
    h2 = jnp.dot(o1, w2_ref[...], preferred_element_type=jnp.float32)
    hs2_ref[...] = h2 * dis


def _tc_last(p_ref, hs2_ref, dis_ref, b2_ref, out_ref):
    z = (p_ref[0] + p_ref[1] + hs2_ref[...]) * dis_ref[...] + b2_ref[...]
    m = jnp.max(z, axis=1, keepdims=True)
    t = z - m
    lse = jnp.log(jnp.sum(jnp.exp(t), axis=1, keepdims=True))
    out_ref[...] = t - lse


def kernel(x, edge_index, W1, b1, W2, b2):
    n, _ = x.shape
    e = edge_index.shape[1]
    epw = e // NW
    ch = epw // C
    npad = ((n + 127) // 128) * 128

    src3 = edge_index[0].reshape(NW, ch, C)
    dst3 = edge_index[1].reshape(NW, ch, C)
    zeros_t = jnp.zeros((npad, 16), jnp.float32)
    ones_t = jnp.ones((C, 16), jnp.float32)

    pdeg = _deg_kernel(npad, ch)(dst3, zeros_t, ones_t)

    B = 1000
    grid = (n // B,)
    bspec_n16 = pl.BlockSpec((B, 16), lambda i: (i, 0))
    bspec_n1 = pl.BlockSpec((B, 1), lambda i: (i, 0))
    bspec_p = pl.BlockSpec((2, B, 16), lambda i: (0, i, 0))
    bspec_b = pl.BlockSpec((1, 16), lambda i: (0, 0))

    hs1, dis = pl.pallas_call(
        _tc_first,
        grid=grid,
        in_specs=[pl.BlockSpec((B, x.shape[1]), lambda i: (i, 0)),
                  pl.BlockSpec(W1.shape, lambda i: (0, 0)),
                  pl.BlockSpec((2, B, 8), lambda i: (0, i, 0))],
        out_specs=(bspec_n16, bspec_n1),
        out_shape=(jax.ShapeDtypeStruct((n, 16), jnp.float32),
                   jax.ShapeDtypeStruct((n, 1), jnp.float32)),
    )(x, W1, pdeg)

    agg = _agg_kernel(npad, ch)
    p1 = agg(hs1, src3, dst3, zeros_t)

    hs2 = pl.pallas_call(
        _tc_mid,
        grid=grid,
        in_specs=[bspec_p, bspec_n16, bspec_n1, bspec_b,
                  pl.BlockSpec(W2.shape, lambda i: (0, 0))],
        out_specs=bspec_n16,
        out_shape=jax.ShapeDtypeStruct((n, 16), jnp.float32),
    )(p1, hs1, dis, b1.reshape(1, 16), W2)

    p2 = agg(hs2, src3, dst3, zeros_t)

    out = pl.pallas_call(
        _tc_last,
        grid=grid,
        in_specs=[bspec_p, bspec_n16, bspec_n1, bspec_b],
        out_specs=bspec_n16,
        out_shape=jax.ShapeDtypeStruct((n, 16), jnp.float32),
    )(p2, hs2, dis, b2.reshape(1, 16))

    return out


# speedup vs baseline: 1.0242x; 1.0242x over previous
import functools

import jax
import jax.numpy as jnp
from jax import lax
from jax.experimental import pallas as pl
from jax.experimental.pallas import tpu as pltpu
from jax.experimental.pallas import tpu_sc as plsc

NC = 2
NS = 16
NW = NC * NS
C = 100
NBUF = 5


def _deg_kernel(n, ch):
    rpt = n // NS

    mesh = plsc.VectorSubcoreMesh(core_axis_name="c", subcore_axis_name="s")

    @functools.partial(
        pl.kernel,
        out_type=jax.ShapeDtypeStruct((NC, n, 8), jnp.float32),
        mesh=mesh,
        compiler_params=pltpu.CompilerParams(use_tc_tiling_on_sc=False),
        scratch_types=[
            pltpu.VMEM((ch, C), jnp.int32),
            pltpu.VMEM((C, 16), jnp.float32),
            pltpu.VMEM_SHARED((n, 16), jnp.float32),
        ],
    )
    def k(dst_hbm, zeros_hbm, ones_hbm, out_hbm, dst_v, ones_v, acc):
        c = lax.axis_index("c")
        s = lax.axis_index("s")
        wid = c * NS + s
        pltpu.sync_copy(dst_hbm.at[wid], dst_v)
        pltpu.sync_copy(ones_hbm, ones_v)
        pltpu.sync_copy(zeros_hbm.at[pl.ds(s * rpt, rpt)], acc.at[pl.ds(s * rpt, rpt)])
        plsc.subcore_barrier()

        def body(g, carry):
            pltpu.sync_copy(ones_v, acc.at[dst_v.at[g]], add=True)
            return carry

        lax.fori_loop(0, ch, body, 0)
        plsc.subcore_barrier()
        pltpu.sync_copy(acc.at[pl.ds(s * rpt, rpt), pl.ds(0, 8)],
                        out_hbm.at[c, pl.ds(s * rpt, rpt)])

    return k


def _agg_kernel(n, ch):
    rpt = n // NS

    mesh = plsc.VectorSubcoreMesh(core_axis_name="c", subcore_axis_name="s")

    @functools.partial(
        pl.kernel,
        out_type=jax.ShapeDtypeStruct((NC, n, 16), jnp.float32),
        mesh=mesh,
        compiler_params=pltpu.CompilerParams(use_tc_tiling_on_sc=False),
        scratch_types=[
            pltpu.VMEM((ch, C), jnp.int32),
            pltpu.VMEM((ch, C), jnp.int32),
            pltpu.VMEM((2, NBUF, C, 16), jnp.float32),
            pltpu.VMEM_SHARED((n, 16), jnp.float32),
            pltpu.SemaphoreType.DMA((2,)),
        ],
    )
    def k(hs_hbm, src_hbm, dst_hbm, zeros_hbm, out_hbm,
          src_v, dst_v, rows_v, acc, gsem):
        c = lax.axis_index("c")
        s = lax.axis_index("s")
        wid = c * NS + s
        pltpu.sync_copy(src_hbm.at[wid], src_v)
        pltpu.sync_copy(dst_hbm.at[wid], dst_v)
        pltpu.sync_copy(zeros_hbm.at[pl.ds(s * rpt, rpt)], acc.at[pl.ds(s * rpt, rpt)])
        plsc.subcore_barrier()

        ngroups = ch // NBUF
        npairs = ngroups // 2

        def fire(grp_idx, p):
            for b in range(NBUF):
                pltpu.async_copy(hs_hbm.at[src_v.at[grp_idx * NBUF + b]],
                                 rows_v.at[p, b], gsem.at[p])

        def drain_scatter(grp_idx, p):
            for b in range(NBUF):
                pltpu.make_async_copy(hs_hbm.at[src_v.at[grp_idx * NBUF + b]],
                                      rows_v.at[p, b], gsem.at[p]).wait()
            for b in range(NBUF):
                pltpu.sync_copy(rows_v.at[p, b],
                                acc.at[dst_v.at[grp_idx * NBUF + b]], add=True)

        fire(0, 0)
        fire(1, 1)

        def pair(k_, carry):
            for p in range(2):
                gi = 2 * k_ + p
                drain_scatter(gi, p)
                fire(gi + 2, p)
            return carry

        lax.fori_loop(0, npairs - 1, pair, 0)
        for p in range(2):
            drain_scatter(2 * (npairs - 1) + p, p)
        plsc.subcore_barrier()
        pltpu.sync_copy(acc.at[pl.ds(s * rpt, rpt)],
                        out_hbm.at[c, pl.ds(s * rpt, rpt)])

    return k


def _tc_first(x_ref, w1_ref, pdeg_ref, hs1_ref, dis_ref):
    n = x_ref.shape[0]
    deg = pdeg_ref[0, :n, 0:1] + pdeg_ref[1, :n, 0:1] + 1.0
    dis = lax.rsqrt(deg)
    h1 = jnp.dot(x_ref[...], w1_ref[...], preferred_element_type=jnp.float32)
    hs1_ref[...] = h1 * dis
    dis_ref[...] = dis


def _tc_mid(p_ref, hs1_ref, dis_ref, b1_ref, w2_ref, hs2_ref):
    n = hs1_ref.shape[0]
    dis = dis_ref[...]
    agg = (p_ref[0, :n, :] + p_ref[1, :n, :] + hs1_ref[...]) * dis + b1_ref[...]
    o1 = jnp.maximum(agg, 0.0)
    h2 = jnp.dot(o1, w2_ref[...], preferred_element_type=jnp.float32)
    hs2_ref[...] = h2 * dis


def _tc_last(p_ref, hs2_ref, dis_ref, b2_ref, out_ref):
    n = hs2_ref.shape[0]
    z = (p_ref[0, :n, :] + p_ref[1, :n, :] + hs2_ref[...]) * dis_ref[...] + b2_ref[...]
    m = jnp.max(z, axis=1, keepdims=True)
    t = z - m
    lse = jnp.log(jnp.sum(jnp.exp(t), axis=1, keepdims=True))
    out_ref[...] = t - lse


def kernel(x, edge_index, W1, b1, W2, b2):
    n, _ = x.shape
    e = edge_index.shape[1]
    epw = e // NW
    ch = epw // C
    npad = ((n + 127) // 128) * 128

    src3 = edge_index[0].reshape(NW, ch, C)
    dst3 = edge_index[1].reshape(NW, ch, C)
    zeros_t = jnp.zeros((npad, 16), jnp.float32)
    ones_t = jnp.ones((C, 16), jnp.float32)

    pdeg = _deg_kernel(npad, ch)(dst3, zeros_t, ones_t)

    hs1, dis = pl.pallas_call(
        _tc_first,
        out_shape=(jax.ShapeDtypeStruct((n, 16), jnp.float32),
                   jax.ShapeDtypeStruct((n, 1), jnp.float32)),
    )(x, W1, pdeg)

    agg = _agg_kernel(npad, ch)
    p1 = agg(hs1, src3, dst3, zeros_t)

    hs2 = pl.pallas_call(
        _tc_mid,
        out_shape=jax.ShapeDtypeStruct((n, 16), jnp.float32),
    )(p1, hs1, dis, b1.reshape(1, 16), W2)

    p2 = agg(hs2, src3, dst3, zeros_t)

    out = pl.pallas_call(
        _tc_last,
        out_shape=jax.ShapeDtypeStruct((n, 16), jnp.float32),
    )(p2, hs2, dis, b2.reshape(1, 16))

    return out


# speedup vs baseline: 1.1005x; 1.0746x over previous
import functools

import jax
import jax.numpy as jnp
from jax import lax
from jax.experimental import pallas as pl
from jax.experimental.pallas import tpu as pltpu
from jax.experimental.pallas import tpu_sc as plsc

NC = 2
NS = 16
NW = NC * NS
C = 100
NBUF = 5


def _deg_kernel(n, ch):
    rpt = n // NS

    mesh = plsc.VectorSubcoreMesh(core_axis_name="c", subcore_axis_name="s")

    @functools.partial(
        pl.kernel,
        out_type=jax.ShapeDtypeStruct((NC, n, 16), jnp.float32),
        mesh=mesh,
        compiler_params=pltpu.CompilerParams(use_tc_tiling_on_sc=False),
        scratch_types=[
            pltpu.VMEM((ch, C), jnp.int32),
            pltpu.VMEM((C, 16), jnp.float32),
            pltpu.VMEM_SHARED((n, 16), jnp.float32),
        ],
    )
    def k(e_hbm, zeros_hbm, ones_hbm, out_hbm, dst_v, ones_v, acc):
        c = lax.axis_index("c")
        s = lax.axis_index("s")
        wid = c * NS + s
        pltpu.sync_copy(e_hbm.at[1, wid], dst_v)
        pltpu.sync_copy(ones_hbm, ones_v)
        pltpu.sync_copy(zeros_hbm.at[pl.ds(s * rpt, rpt)], acc.at[pl.ds(s * rpt, rpt)])
        plsc.subcore_barrier()

        def body(g, carry):
            pltpu.sync_copy(ones_v, acc.at[dst_v.at[g]], add=True)
            return carry

        lax.fori_loop(0, ch, body, 0)
        plsc.subcore_barrier()
        pltpu.sync_copy(acc.at[pl.ds(s * rpt, rpt)],
                        out_hbm.at[c, pl.ds(s * rpt, rpt)])

    return k


def _agg_kernel(n, ch):
    rpt = n // NS

    mesh = plsc.VectorSubcoreMesh(core_axis_name="c", subcore_axis_name="s")

    @functools.partial(
        pl.kernel,
        out_type=jax.ShapeDtypeStruct((NC, n, 16), jnp.float32),
        mesh=mesh,
        compiler_params=pltpu.CompilerParams(use_tc_tiling_on_sc=False),
        scratch_types=[
            pltpu.VMEM((ch, C), jnp.int32),
            pltpu.VMEM((ch, C), jnp.int32),
            pltpu.VMEM((2, NBUF, C, 16), jnp.float32),
            pltpu.VMEM_SHARED((n, 16), jnp.float32),
            pltpu.SemaphoreType.DMA((2,)),
        ],
    )
    def k(hs_hbm, e_hbm, zeros_hbm, out_hbm,
          src_v, dst_v, rows_v, acc, gsem):
        c = lax.axis_index("c")
        s = lax.axis_index("s")
        wid = c * NS + s
        pltpu.sync_copy(e_hbm.at[0, wid], src_v)
        pltpu.sync_copy(e_hbm.at[1, wid], dst_v)
        pltpu.sync_copy(zeros_hbm.at[pl.ds(s * rpt, rpt)], acc.at[pl.ds(s * rpt, rpt)])
        plsc.subcore_barrier()

        ngroups = ch // NBUF
        npairs = ngroups // 2

        def fire(grp_idx, p):
            for b in range(NBUF):
                pltpu.async_copy(hs_hbm.at[src_v.at[grp_idx * NBUF + b]],
                                 rows_v.at[p, b], gsem.at[p])

        def drain_scatter(grp_idx, p):
            for b in range(NBUF):
                pltpu.make_async_copy(hs_hbm.at[src_v.at[grp_idx * NBUF + b]],
                                      rows_v.at[p, b], gsem.at[p]).wait()
            for b in range(NBUF):
                pltpu.sync_copy(rows_v.at[p, b],
                                acc.at[dst_v.at[grp_idx * NBUF + b]], add=True)

        fire(0, 0)
        fire(1, 1)

        def pair(k_, carry):
            for p in range(2):
                gi = 2 * k_ + p
                drain_scatter(gi, p)
                fire(gi + 2, p)
            return carry

        lax.fori_loop(0, npairs - 1, pair, 0)
        for p in range(2):
            drain_scatter(2 * (npairs - 1) + p, p)
        plsc.subcore_barrier()
        pltpu.sync_copy(acc.at[pl.ds(s * rpt, rpt)],
                        out_hbm.at[c, pl.ds(s * rpt, rpt)])

    return k


def _tc_first(x_ref, w1_ref, pdeg_ref, hs1_ref, dis_ref):
    n = x_ref.shape[0]
    deg = pdeg_ref[0, :n, 0:1] + pdeg_ref[1, :n, 0:1] + 1.0
    dis = lax.rsqrt(deg)
    h1 = jnp.dot(x_ref[...], w1_ref[...], preferred_element_type=jnp.float32)
    hs1_ref[...] = h1 * dis
    dis_ref[...] = dis


def _tc_mid(p_ref, hs1_ref, dis_ref, b1_ref, w2_ref, hs2_ref):
    n = hs1_ref.shape[0]
    dis = dis_ref[...]
    agg = (p_ref[0, :n, :] + p_ref[1, :n, :] + hs1_ref[...]) * dis + b1_ref[...]
    o1 = jnp.maximum(agg, 0.0)
    h2 = jnp.dot(o1, w2_ref[...], preferred_element_type=jnp.float32)
    hs2_ref[...] = h2 * dis


def _tc_last(p_ref, hs2_ref, dis_ref, b2_ref, out_ref):
    n = hs2_ref.shape[0]
    z = (p_ref[0, :n, :] + p_ref[1, :n, :] + hs2_ref[...]) * dis_ref[...] + b2_ref[...]
    m = jnp.max(z, axis=1, keepdims=True)
    t = z - m
    lse = jnp.log(jnp.sum(jnp.exp(t), axis=1, keepdims=True))
    out_ref[...] = t - lse


def kernel(x, edge_index, W1, b1, W2, b2):
    n, _ = x.shape
    e = edge_index.shape[1]
    epw = e // NW
    ch = epw // C
    npad = ((n + 127) // 128) * 128

    e4 = edge_index.reshape(2, NW, ch, C)
    zeros_t = jnp.zeros((npad, 16), jnp.float32)
    ones_t = jnp.ones((C, 16), jnp.float32)

    pdeg = _deg_kernel(npad, ch)(e4, zeros_t, ones_t)

    hs1, dis = pl.pallas_call(
        _tc_first,
        out_shape=(jax.ShapeDtypeStruct((n, 16), jnp.float32),
                   jax.ShapeDtypeStruct((n, 1), jnp.float32)),
    )(x, W1, pdeg)

    agg = _agg_kernel(npad, ch)
    p1 = agg(hs1, e4, zeros_t)

    hs2 = pl.pallas_call(
        _tc_mid,
        out_shape=jax.ShapeDtypeStruct((n, 16), jnp.float32),
    )(p1, hs1, dis, b1.reshape(1, 16), W2)

    p2 = agg(hs2, e4, zeros_t)

    out = pl.pallas_call(
        _tc_last,
        out_shape=jax.ShapeDtypeStruct((n, 16), jnp.float32),
    )(p2, hs2, dis, b2.reshape(1, 16))

    return out


# speedup vs baseline: 1.5734x; 1.4296x over previous
import functools

import jax
import jax.numpy as jnp
from jax import lax
from jax.experimental import pallas as pl
from jax.experimental.pallas import tpu as pltpu
from jax.experimental.pallas import tpu_sc as plsc

NC = 2
NS = 16
NW = NC * NS
C = 100
NBUF = 5


def _deg_kernel(n, ch):
    rpt = n // NS

    mesh = plsc.VectorSubcoreMesh(core_axis_name="c", subcore_axis_name="s")

    @functools.partial(
        pl.kernel,
        out_type=jax.ShapeDtypeStruct((NC, n, 16), jnp.float32),
        mesh=mesh,
        compiler_params=pltpu.CompilerParams(use_tc_tiling_on_sc=False),
        scratch_types=[
            pltpu.VMEM((ch, C), jnp.int32),
            pltpu.VMEM((C, 16), jnp.float32),
            pltpu.VMEM_SHARED((n, 16), jnp.float32),
        ],
    )
    def k(e_hbm, zeros_hbm, ones_hbm, out_hbm, dst_v, ones_v, acc):
        c = lax.axis_index("c")
        s = lax.axis_index("s")
        wid = c * NS + s
        pltpu.sync_copy(e_hbm.at[1, wid], dst_v)
        pltpu.sync_copy(ones_hbm, ones_v)
        pltpu.sync_copy(zeros_hbm.at[pl.ds(s * rpt, rpt)], acc.at[pl.ds(s * rpt, rpt)])
        plsc.subcore_barrier()

        def body(g, carry):
            pltpu.sync_copy(ones_v, acc.at[dst_v.at[g]], add=True)
            return carry

        lax.fori_loop(0, ch, body, 0)
        plsc.subcore_barrier()
        pltpu.sync_copy(acc.at[pl.ds(s * rpt, rpt)],
                        out_hbm.at[c, pl.ds(s * rpt, rpt)])

    return k


def _agg_kernel(n, ch):
    rpt = n // NS

    mesh = plsc.VectorSubcoreMesh(core_axis_name="c", subcore_axis_name="s")

    @functools.partial(
        pl.kernel,
        out_type=jax.ShapeDtypeStruct((NC, n, 16), jnp.float32),
        mesh=mesh,
        compiler_params=pltpu.CompilerParams(use_tc_tiling_on_sc=False),
        scratch_types=[
            pltpu.VMEM((ch, C), jnp.int32),
            pltpu.VMEM((ch, C), jnp.int32),
            pltpu.VMEM((2, NBUF, C, 16), jnp.float32),
            pltpu.VMEM_SHARED((n, 16), jnp.float32),
            pltpu.SemaphoreType.DMA((2,)),
        ],
    )
    def k(hs_hbm, e_hbm, zeros_hbm, out_hbm,
          src_v, dst_v, rows_v, acc, gsem):
        c = lax.axis_index("c")
        s = lax.axis_index("s")
        wid = c * NS + s
        pltpu.sync_copy(e_hbm.at[0, wid], src_v)
        pltpu.sync_copy(e_hbm.at[1, wid], dst_v)
        pltpu.sync_copy(zeros_hbm.at[pl.ds(s * rpt, rpt)], acc.at[pl.ds(s * rpt, rpt)])
        plsc.subcore_barrier()

        ngroups = ch // NBUF
        npairs = ngroups // 2

        def fire(grp_idx, p):
            for b in range(NBUF):
                pltpu.async_copy(hs_hbm.at[src_v.at[grp_idx * NBUF + b]],
                                 rows_v.at[p, b], gsem.at[p])

        def drain_scatter(grp_idx, p):
            for b in range(NBUF):
                pltpu.make_async_copy(hs_hbm.at[src_v.at[grp_idx * NBUF + b]],
                                      rows_v.at[p, b], gsem.at[p]).wait()
            for b in range(NBUF):
                pltpu.sync_copy(rows_v.at[p, b],
                                acc.at[dst_v.at[grp_idx * NBUF + b]], add=True)

        fire(0, 0)
        fire(1, 1)

        def pair(k_, carry):
            for p in range(2):
                gi = 2 * k_ + p
                drain_scatter(gi, p)
                fire(gi + 2, p)
            return carry

        lax.fori_loop(0, npairs - 1, pair, 0)
        for p in range(2):
            drain_scatter(2 * (npairs - 1) + p, p)
        plsc.subcore_barrier()
        pltpu.sync_copy(acc.at[pl.ds(s * rpt, rpt)],
                        out_hbm.at[c, pl.ds(s * rpt, rpt)])

    return k


def _tc_first(x8_ref, bd1_ref, pdegp_ref, hs1_ref, disp_ref):
    m = hs1_ref.shape[0]
    deg = pdegp_ref[0, :m, :] + pdegp_ref[1, :m, :] + 1.0
    dis = lax.rsqrt(deg)
    h1 = jnp.dot(x8_ref[...], bd1_ref[...], preferred_element_type=jnp.float32)
    hs1_ref[...] = h1 * dis
    disp_ref[...] = dis


def _tc_mid(pp_ref, hs1_ref, disp_ref, b1t_ref, bd2_ref, hs2_ref):
    m = hs1_ref.shape[0]
    dis = disp_ref[...]
    agg = (pp_ref[0, :m, :] + pp_ref[1, :m, :] + hs1_ref[...]) * dis + b1t_ref[...]
    o1 = jnp.maximum(agg, 0.0)
    h2 = jnp.dot(o1, bd2_ref[...], preferred_element_type=jnp.float32)
    hs2_ref[...] = h2 * dis


def _tc_last(pp_ref, hs2_ref, disp_ref, b2t_ref, g_ref, out_ref):
    m = hs2_ref.shape[0]
    z = (pp_ref[0, :m, :] + pp_ref[1, :m, :] + hs2_ref[...]) * disp_ref[...] + b2t_ref[...]
    mx = jnp.max(z, axis=1, keepdims=True)
    t = z - mx
    p = jnp.dot(jnp.exp(t), g_ref[...], preferred_element_type=jnp.float32)
    out_ref[...] = t - jnp.log(p)


def kernel(x, edge_index, W1, b1, W2, b2):
    n, _ = x.shape
    e = edge_index.shape[1]
    epw = e // NW
    ch = epw // C
    npad = ((n + 127) // 128) * 128

    e4 = edge_index.reshape(2, NW, ch, C)
    zeros_t = jnp.zeros((npad, 16), jnp.float32)
    ones_t = jnp.ones((C, 16), jnp.float32)

    m = n * 16 // 128
    mp = npad * 16 // 128
    eye8 = jnp.eye(8, dtype=jnp.float32)
    bd1 = jnp.kron(eye8, W1)
    bd2 = jnp.kron(eye8, W2)
    gsum = jnp.kron(eye8, jnp.ones((16, 16), jnp.float32))
    b1t = jnp.tile(b1, 8).reshape(1, 128)
    b2t = jnp.tile(b2, 8).reshape(1, 128)

    pdeg = _deg_kernel(npad, ch)(e4, zeros_t, ones_t)

    hs1, dis = pl.pallas_call(
        _tc_first,
        out_shape=(jax.ShapeDtypeStruct((m, 128), jnp.float32),
                   jax.ShapeDtypeStruct((m, 128), jnp.float32)),
    )(x.reshape(m, 8 * x.shape[1]), bd1, pdeg.reshape(NC, mp, 128))

    agg = _agg_kernel(npad, ch)
    p1 = agg(hs1.reshape(n, 16), e4, zeros_t)

    hs2 = pl.pallas_call(
        _tc_mid,
        out_shape=jax.ShapeDtypeStruct((m, 128), jnp.float32),
    )(p1.reshape(NC, mp, 128), hs1, dis, b1t, bd2)

    p2 = agg(hs2.reshape(n, 16), e4, zeros_t)

    out = pl.pallas_call(
        _tc_last,
        out_shape=jax.ShapeDtypeStruct((m, 128), jnp.float32),
    )(p2.reshape(NC, mp, 128), hs2, dis, b2t, gsum)

    return out.reshape(n, 16)


# speedup vs baseline: 1.7782x; 1.1302x over previous
import functools

import jax
import jax.numpy as jnp
from jax import lax
from jax.experimental import pallas as pl
from jax.experimental.pallas import tpu as pltpu
from jax.experimental.pallas import tpu_sc as plsc

NC = 2
NS = 16
NW = NC * NS
C = 125
NBUF = 10


def _deg_kernel(n, ch):
    rpt = n // NS

    mesh = plsc.VectorSubcoreMesh(core_axis_name="c", subcore_axis_name="s")

    @functools.partial(
        pl.kernel,
        out_type=jax.ShapeDtypeStruct((NC, n, 16), jnp.float32),
        mesh=mesh,
        compiler_params=pltpu.CompilerParams(use_tc_tiling_on_sc=False),
        scratch_types=[
            pltpu.VMEM((ch, C), jnp.int32),
            pltpu.VMEM((C, 16), jnp.float32),
            pltpu.VMEM_SHARED((n, 16), jnp.float32),
            pltpu.SemaphoreType.DMA,
        ],
    )
    def k(e_hbm, zeros_hbm, ones_hbm, out_hbm, dst_v, ones_v, acc, ssem):
        c = lax.axis_index("c")
        s = lax.axis_index("s")
        wid = c * NS + s
        pltpu.sync_copy(e_hbm.at[1, wid], dst_v)
        pltpu.sync_copy(ones_hbm, ones_v)
        pltpu.sync_copy(zeros_hbm.at[pl.ds(s * rpt, rpt)], acc.at[pl.ds(s * rpt, rpt)])
        plsc.subcore_barrier()

        def body(g, carry):
            pltpu.async_copy(ones_v, acc.at[dst_v.at[g]], ssem)
            return carry

        lax.fori_loop(0, ch, body, 0)
        for g in range(ch):
            pltpu.make_async_copy(ones_v, acc.at[dst_v.at[g]], ssem).wait()
        plsc.subcore_barrier()
        pltpu.sync_copy(acc.at[pl.ds(s * rpt, rpt)],
                        out_hbm.at[c, pl.ds(s * rpt, rpt)])

    return k


def _agg_kernel(n, ch):
    rpt = n // NS

    mesh = plsc.VectorSubcoreMesh(core_axis_name="c", subcore_axis_name="s")

    @functools.partial(
        pl.kernel,
        out_type=jax.ShapeDtypeStruct((NC, n, 16), jnp.float32),
        mesh=mesh,
        compiler_params=pltpu.CompilerParams(use_tc_tiling_on_sc=False),
        scratch_types=[
            pltpu.VMEM((ch, C), jnp.int32),
            pltpu.VMEM((ch, C), jnp.int32),
            pltpu.VMEM((2, NBUF, C, 16), jnp.float32),
            pltpu.VMEM_SHARED((n, 16), jnp.float32),
            pltpu.SemaphoreType.DMA((2,)),
        ],
    )
    def k(hs_hbm, e_hbm, zeros_hbm, out_hbm,
          src_v, dst_v, rows_v, acc, gsem):
        c = lax.axis_index("c")
        s = lax.axis_index("s")
        wid = c * NS + s
        pltpu.sync_copy(e_hbm.at[0, wid], src_v)
        pltpu.sync_copy(e_hbm.at[1, wid], dst_v)
        pltpu.sync_copy(zeros_hbm.at[pl.ds(s * rpt, rpt)], acc.at[pl.ds(s * rpt, rpt)])
        plsc.subcore_barrier()

        ngroups = ch // NBUF
        npairs = ngroups // 2

        def fire(grp_idx, p):
            for b in range(NBUF):
                pltpu.async_copy(hs_hbm.at[src_v.at[grp_idx * NBUF + b]],
                                 rows_v.at[p, b], gsem.at[p])

        def drain_scatter(grp_idx, p):
            for b in range(NBUF):
                pltpu.make_async_copy(hs_hbm.at[src_v.at[grp_idx * NBUF + b]],
                                      rows_v.at[p, b], gsem.at[p]).wait()
            for b in range(NBUF):
                pltpu.sync_copy(rows_v.at[p, b],
                                acc.at[dst_v.at[grp_idx * NBUF + b]], add=True)

        fire(0, 0)
        fire(1, 1)

        def pair(k_, carry):
            for p in range(2):
                gi = 2 * k_ + p
                drain_scatter(gi, p)
                fire(gi + 2, p)
            return carry

        lax.fori_loop(0, npairs - 1, pair, 0)
        for p in range(2):
            drain_scatter(2 * (npairs - 1) + p, p)
        plsc.subcore_barrier()
        pltpu.sync_copy(acc.at[pl.ds(s * rpt, rpt)],
                        out_hbm.at[c, pl.ds(s * rpt, rpt)])

    return k


def _tc_first(x8_ref, bd1_ref, pdegp_ref, hs1_ref, disp_ref):
    m = hs1_ref.shape[0]
    deg = pdegp_ref[0, :m, :] + pdegp_ref[1, :m, :] + 1.0
    dis = lax.rsqrt(deg)
    h1 = jnp.dot(x8_ref[...], bd1_ref[...], preferred_element_type=jnp.float32)
    hs1_ref[...] = h1 * dis
    disp_ref[...] = dis


def _tc_mid(pp_ref, hs1_ref, disp_ref, b1t_ref, bd2_ref, hs2_ref):
    m = hs1_ref.shape[0]
    dis = disp_ref[...]
    agg = (pp_ref[0, :m, :] + pp_ref[1, :m, :] + hs1_ref[...]) * dis + b1t_ref[...]
    o1 = jnp.maximum(agg, 0.0)
    h2 = jnp.dot(o1, bd2_ref[...], preferred_element_type=jnp.float32)
    hs2_ref[...] = h2 * dis


def _tc_last(pp_ref, hs2_ref, disp_ref, b2t_ref, g_ref, out_ref):
    m = hs2_ref.shape[0]
    z = (pp_ref[0, :m, :] + pp_ref[1, :m, :] + hs2_ref[...]) * disp_ref[...] + b2t_ref[...]
    mx = jnp.max(z, axis=1, keepdims=True)
    t = z - mx
    p = jnp.dot(jnp.exp(t), g_ref[...], preferred_element_type=jnp.float32)
    out_ref[...] = t - jnp.log(p)


def kernel(x, edge_index, W1, b1, W2, b2):
    n, _ = x.shape
    e = edge_index.shape[1]
    epw = e // NW
    ch = epw // C
    npad = ((n + 127) // 128) * 128

    e4 = edge_index.reshape(2, NW, ch, C)
    zeros_t = jnp.zeros((npad, 16), jnp.float32)
    ones_t = jnp.ones((C, 16), jnp.float32)

    m = n * 16 // 128
    mp = npad * 16 // 128
    eye8 = jnp.eye(8, dtype=jnp.float32)
    bd1 = jnp.kron(eye8, W1)
    bd2 = jnp.kron(eye8, W2)
    gsum = jnp.kron(eye8, jnp.ones((16, 16), jnp.float32))
    b1t = jnp.tile(b1, 8).reshape(1, 128)
    b2t = jnp.tile(b2, 8).reshape(1, 128)

    pdeg = _deg_kernel(npad, ch)(e4, zeros_t, ones_t)

    hs1, dis = pl.pallas_call(
        _tc_first,
        out_shape=(jax.ShapeDtypeStruct((m, 128), jnp.float32),
                   jax.ShapeDtypeStruct((m, 128), jnp.float32)),
    )(x.reshape(m, 8 * x.shape[1]), bd1, pdeg.reshape(NC, mp, 128))

    agg = _agg_kernel(npad, ch)
    p1 = agg(hs1.reshape(n, 16), e4, zeros_t)

    hs2 = pl.pallas_call(
        _tc_mid,
        out_shape=jax.ShapeDtypeStruct((m, 128), jnp.float32),
    )(p1.reshape(NC, mp, 128), hs1, dis, b1t, bd2)

    p2 = agg(hs2.reshape(n, 16), e4, zeros_t)

    out = pl.pallas_call(
        _tc_last,
        out_shape=jax.ShapeDtypeStruct((m, 128), jnp.float32),
    )(p2.reshape(NC, mp, 128), hs2, dis, b2t, gsum)

    return out.reshape(n, 16)
